# baseline (speedup 1.0000x reference)
import functools
import math

import jax
import jax.numpy as jnp
from jax.experimental import pallas as pl
from jax.experimental.pallas import tpu as pltpu

_B, _N, _C = 64, 1024, 768
_Q = _C // 4
_LN10K = math.log(10000.0)


def _pe_body(hdr_ref, coords_ref, pe_ref, *, gw_static):
    gw = hdr_ref[0]
    gh = hdr_ref[1]
    c = coords_ref[...]
    idx = (c[:, 1] * gw + c[:, 0]) % (gw * gh)
    pos_i = (idx // gw_static).astype(jnp.float32)[:, None]
    pos_j = (idx % gw_static).astype(jnp.float32)[:, None]
    d = jax.lax.broadcasted_iota(jnp.int32, (_N, _C), 1)
    q = d // _Q
    r = (d % _Q).astype(jnp.float32)
    omega = jnp.exp(r * (-_LN10K / _Q))
    base = jnp.where(q < 2, pos_j, pos_i)
    phase = jnp.where((q % 2) == 1, 0.5 * math.pi, 0.0)
    pe_ref[...] = jnp.sin(base * omega + phase)


def _add_body(pe_ref, x_ref, o_ref):
    o_ref[...] = x_ref[...] + pe_ref[...][None, :, :]


@jax.jit
def kernel(x, pos):
    B, N, C = x.shape
    gh_static = int(math.isqrt(pos.shape[0] - 1))
    gw_static = (pos.shape[0] - 1) // gh_static
    hdr = pos[0]
    coords = pos[1:]

    pe = pl.pallas_call(
        functools.partial(_pe_body, gw_static=gw_static),
        out_shape=jax.ShapeDtypeStruct((N, C), jnp.float32),
        in_specs=[
            pl.BlockSpec(memory_space=pltpu.SMEM),
            pl.BlockSpec(memory_space=pltpu.VMEM),
        ],
        out_specs=pl.BlockSpec(memory_space=pltpu.VMEM),
    )(hdr, coords)

    out = pl.pallas_call(
        _add_body,
        grid=(B,),
        out_shape=jax.ShapeDtypeStruct((B, N, C), jnp.float32),
        in_specs=[
            pl.BlockSpec((N, C), lambda b: (0, 0)),
            pl.BlockSpec((1, N, C), lambda b: (b, 0, 0)),
        ],
        out_specs=pl.BlockSpec((1, N, C), lambda b: (b, 0, 0)),
        compiler_params=pltpu.CompilerParams(
            dimension_semantics=("parallel",),
        ),
    )(pe, x)
    return out


# speedup vs baseline: 1.0282x; 1.0282x over previous
import functools
import math

import jax
import jax.numpy as jnp
import numpy as np
from jax.experimental import pallas as pl
from jax.experimental.pallas import tpu as pltpu

_B, _N, _C = 64, 1024, 768
_Q = _C // 4

_d = np.arange(_C)
_q = _d // _Q
_omega = (10000.0 ** (-((_d % _Q) / float(_Q)))).astype(np.float32)
_WJ = np.where(_q < 2, _omega, 0.0).astype(np.float32).reshape(1, _C)
_WI = np.where(_q >= 2, _omega, 0.0).astype(np.float32).reshape(1, _C)
_PHASE = np.where(_q % 2 == 1, 0.5 * np.pi, 0.0).astype(np.float32).reshape(1, _C)


_ROWC = np.concatenate([_WJ, _WI, _PHASE], axis=0)


def _pe_body(hdr_ref, coords_ref, rowc_ref, pe_ref, *, gw_static):
    gw = hdr_ref[0]
    gh = hdr_ref[1]
    c = coords_ref[...]
    idx = (c[:, 1] * gw + c[:, 0]) % (gw * gh)
    pos_i = (idx // gw_static).astype(jnp.float32)[:, None]
    pos_j = (idx % gw_static).astype(jnp.float32)[:, None]
    ang = (pos_j * rowc_ref[0:1, :] + pos_i * rowc_ref[1:2, :]
           + rowc_ref[2:3, :])
    pe_ref[...] = jnp.sin(ang)


def _add_body(pe_ref, x_ref, o_ref):
    o_ref[...] = x_ref[...] + pe_ref[...][None, :, :]


@jax.jit
def kernel(x, pos):
    B, N, C = x.shape
    gh_static = int(math.isqrt(pos.shape[0] - 1))
    gw_static = (pos.shape[0] - 1) // gh_static
    hdr = pos[0]
    coords = pos[1:]

    pe = pl.pallas_call(
        functools.partial(_pe_body, gw_static=gw_static),
        out_shape=jax.ShapeDtypeStruct((N, C), jnp.float32),
        in_specs=[
            pl.BlockSpec(memory_space=pltpu.SMEM),
            pl.BlockSpec(memory_space=pltpu.VMEM),
            pl.BlockSpec(memory_space=pltpu.VMEM),
        ],
        out_specs=pl.BlockSpec(memory_space=pltpu.VMEM),
    )(hdr, coords, jnp.asarray(_ROWC))

    bb = 2
    out = pl.pallas_call(
        _add_body,
        grid=(B // bb,),
        out_shape=jax.ShapeDtypeStruct((B, N, C), jnp.float32),
        in_specs=[
            pl.BlockSpec((N, C), lambda b: (0, 0)),
            pl.BlockSpec((bb, N, C), lambda b: (b, 0, 0)),
        ],
        out_specs=pl.BlockSpec((bb, N, C), lambda b: (b, 0, 0)),
        compiler_params=pltpu.CompilerParams(
            dimension_semantics=("parallel",),
        ),
    )(pe, x)
    return out


# speedup vs baseline: 1.0381x; 1.0096x over previous
import functools
import math

import jax
import jax.numpy as jnp
import numpy as np
from jax.experimental import pallas as pl
from jax.experimental.pallas import tpu as pltpu

_B, _N, _C = 64, 1024, 768
_Q = _C // 4

_d = np.arange(_C)
_q = _d // _Q
_omega = (10000.0 ** (-((_d % _Q) / float(_Q)))).astype(np.float32)
_WJ = np.where(_q < 2, _omega, 0.0).astype(np.float32).reshape(1, _C)
_WI = np.where(_q >= 2, _omega, 0.0).astype(np.float32).reshape(1, _C)
_PHASE = np.where(_q % 2 == 1, 0.5 * np.pi, 0.0).astype(np.float32).reshape(1, _C)


_ROWC = np.concatenate([_WJ, _WI, _PHASE], axis=0)


def _pe_body(hdr_ref, coords_ref, rowc_ref, pe_ref, *, gw_static):
    gw = hdr_ref[0]
    gh = hdr_ref[1]
    c = coords_ref[...]
    idx = (c[:, 1] * gw + c[:, 0]) % (gw * gh)
    pos_i = (idx // gw_static).astype(jnp.float32)[:, None]
    pos_j = (idx % gw_static).astype(jnp.float32)[:, None]
    ang = (pos_j * rowc_ref[0:1, :] + pos_i * rowc_ref[1:2, :]
           + rowc_ref[2:3, :])
    pe_ref[...] = jnp.sin(ang)


def _add_body(pe_ref, x_ref, o_ref):
    o_ref[...] = x_ref[...] + pe_ref[...][None, :, :]


@jax.jit
def kernel(x, pos):
    B, N, C = x.shape
    gh_static = int(math.isqrt(pos.shape[0] - 1))
    gw_static = (pos.shape[0] - 1) // gh_static
    hdr = pos[0]
    coords = pos[1:]

    pe = pl.pallas_call(
        functools.partial(_pe_body, gw_static=gw_static),
        out_shape=jax.ShapeDtypeStruct((N, C), jnp.float32),
        in_specs=[
            pl.BlockSpec(memory_space=pltpu.SMEM),
            pl.BlockSpec(memory_space=pltpu.VMEM),
            pl.BlockSpec(memory_space=pltpu.VMEM),
        ],
        out_specs=pl.BlockSpec(memory_space=pltpu.VMEM),
    )(hdr, coords, jnp.asarray(_ROWC))

    bb = 4
    out = pl.pallas_call(
        _add_body,
        grid=(B // bb,),
        out_shape=jax.ShapeDtypeStruct((B, N, C), jnp.float32),
        in_specs=[
            pl.BlockSpec((N, C), lambda b: (0, 0)),
            pl.BlockSpec((bb, N, C), lambda b: (b, 0, 0)),
        ],
        out_specs=pl.BlockSpec((bb, N, C), lambda b: (b, 0, 0)),
        compiler_params=pltpu.CompilerParams(
            dimension_semantics=("parallel",),
        ),
    )(pe, x)
    return out
